# baseline (speedup 1.0000x reference)
import functools

import jax
import jax.numpy as jnp
from jax import lax
from jax.experimental import pallas as pl
from jax.experimental.pallas import tpu as pltpu
from jax.experimental.pallas import tpu_sc as plsc

_B, _CIN, _H, _W = 4, 128, 32, 32
_COUT = 512
_K = 9
_GADD, _RADD = 2, 2
_REGION = (8, 8)
_MIN_SIGMA = 0.05
_SIGMA_SCALE = 0.05
_SIGMA_BOOST = 2.0
_MMULT = 1.0
_HIDDEN = _CIN * 4
_VS = 4 + _GADD + _RADD

_NW = 32
_G = _B * _H * _W * _K
_GPW = _G // _NW
_BG = 16
_NBLK = _GPW // _BG
_NROW = _B * _H * _W

_INTERPRET = False


def _coords_hw(h, w):
    ci = jnp.arange(h, dtype=jnp.float32) / (h - 1)
    cj = jnp.arange(w, dtype=jnp.float32) / (w - 1)
    return jnp.stack([
        jnp.broadcast_to(ci[:, None], (h, w)),
        jnp.broadcast_to(cj[None, :], (h, w)),
    ], axis=0)


def _random_index_constants(b, h, w, k):
    kg = jax.random.key(42)
    kg1, kg2 = jax.random.split(kg)
    glob = jnp.stack([
        jax.random.randint(jax.random.fold_in(kg1, 0), (b, h, w, k, _GADD), 0, h),
        jax.random.randint(jax.random.fold_in(kg1, 1), (b, h, w, k, _GADD), 0, w),
    ], axis=-1).astype(jnp.int32)
    roff = jnp.stack([
        jax.random.randint(jax.random.fold_in(kg2, 0), (b, h, w, k, _RADD), 0, _REGION[0]) - _REGION[0] // 2,
        jax.random.randint(jax.random.fold_in(kg2, 1), (b, h, w, k, _RADD), 0, _REGION[1]) - _REGION[1] // 2,
    ], axis=-1).astype(jnp.int32)
    return glob, roff


def _bcast_lane(vec, lane):
    idx = jnp.full((16, 1), lane, jnp.int32)
    dn = lax.GatherDimensionNumbers(
        offset_dims=(), collapsed_slice_dims=(0,), start_index_map=(0,))
    return lax.gather(vec, idx, dn, (1,),
                      mode=lax.GatherScatterMode.PROMISE_IN_BOUNDS)


def _sc_gather_combine(xt, idxs, wts):
    @functools.partial(
        pl.kernel,
        out_type=jax.ShapeDtypeStruct((_G, _CIN), jnp.float32),
        mesh=plsc.VectorSubcoreMesh(core_axis_name="c", subcore_axis_name="s"),
        scratch_types=[
            pltpu.VMEM((_NBLK, 128), jnp.int32),
            pltpu.VMEM((_NBLK, 128), jnp.float32),
            pltpu.VMEM((2, 128, _CIN), jnp.float32),
            pltpu.VMEM((2, _BG, _CIN), jnp.float32),
            pltpu.SemaphoreType.DMA,
            pltpu.SemaphoreType.DMA,
            pltpu.SemaphoreType.DMA,
            pltpu.SemaphoreType.DMA,
        ],
    )
    def k(xt_hbm, idx_hbm, w_hbm, feat_hbm, idx_v, w_v, rows_v, out_v,
          gsem0, gsem1, osem0, osem1):
        wid = lax.axis_index("s") * 2 + lax.axis_index("c")
        pltpu.sync_copy(idx_hbm.at[wid], idx_v)
        pltpu.sync_copy(w_hbm.at[wid], w_v)

        def combine(buf, blk):
            def gbody(pr, carry):
                wvec = w_v[blk, pl.ds(pr * 16, 16)]
                for half in range(2):
                    row0 = (pr * 2 + half) * _VS
                    wb = [_bcast_lane(wvec, half * _VS + vs) for vs in range(_VS)]
                    for cp in range(_CIN // 16):
                        acc = None
                        for vs in range(_VS):
                            r = rows_v[buf, row0 + vs, pl.ds(cp * 16, 16)]
                            acc = wb[vs] * r if acc is None else acc + wb[vs] * r
                        out_v[buf, pr * 2 + half, pl.ds(cp * 16, 16)] = acc
                return carry
            lax.fori_loop(0, _BG // 2, gbody, 0)

        def gather(buf, blk, sem):
            return pltpu.async_copy(xt_hbm.at[idx_v.at[blk]], rows_v.at[buf], sem)

        def wait_gather(buf, sem):
            pltpu.make_async_copy(xt_hbm.at[idx_v.at[0]], rows_v.at[buf], sem).wait()

        def put_out(buf, blk, sem):
            base = pl.multiple_of(wid * _GPW + blk * _BG, 8)
            return pltpu.async_copy(out_v.at[buf],
                                    feat_hbm.at[pl.ds(base, _BG)], sem)

        def wait_out(buf, sem):
            pltpu.make_async_copy(out_v.at[buf],
                                  feat_hbm.at[pl.ds(0, _BG)], sem).wait()

        gather(0, 0, gsem0)

        def body(i, carry):
            blk0 = i * 2
            blk1 = i * 2 + 1
            gather(1, blk1, gsem1)
            wait_gather(0, gsem0)

            @pl.when(i >= 1)
            def _():
                wait_out(0, osem0)
            combine(0, blk0)
            put_out(0, blk0, osem0)

            @pl.when(i + 1 < _NBLK // 2)
            def _():
                gather(0, blk0 + 2, gsem0)
            wait_gather(1, gsem1)

            @pl.when(i >= 1)
            def _():
                wait_out(1, osem1)
            combine(1, blk1)
            put_out(1, blk1, osem1)
            return carry

        lax.fori_loop(0, _NBLK // 2, body, 0)
        wait_out(0, osem0)
        wait_out(1, osem1)

    return k(xt, idxs, wts)


def _matmul_kernel(wu_ref, feat_ref, bu_ref, out_ref):
    acc = lax.dot_general(wu_ref[...], feat_ref[...],
                          (((1,), (1,)), ((), ())),
                          preferred_element_type=jnp.float32)
    out_ref[...] = (acc + bu_ref[...])[None]


def _unify_matmul(wu_b, feat, bu):
    cout, kc = wu_b.shape
    p = _H * _W
    grid = (_B,)
    return pl.pallas_call(
        _matmul_kernel,
        grid=grid,
        in_specs=[
            pl.BlockSpec((cout, kc), lambda i: (0, 0)),
            pl.BlockSpec((p, kc), lambda i: (i, 0)),
            pl.BlockSpec((cout, 1), lambda i: (0, 0)),
        ],
        out_specs=pl.BlockSpec((1, cout, p), lambda i: (i, 0, 0)),
        out_shape=jax.ShapeDtypeStruct((_B, cout, p), jnp.float32),
        interpret=_INTERPRET,
    )(wu_b, feat, bu.reshape(cout, 1))


def kernel(x, W1, b1, W2, b2, Wu, bu):
    b, c, h, w = x.shape
    k = _K
    coords = _coords_hw(h, w)
    hw = jnp.array([h, w], dtype=jnp.float32)

    mids = coords * (hw - 1.0)[:, None, None]
    mids = mids.transpose(1, 2, 0).reshape(h * w, 1, 2)

    inp = coords.transpose(1, 2, 0).reshape(h * w, 2)
    hdn = jax.nn.relu(inp @ W1.T + b1)
    params = hdn @ W2.T + b2
    means = params[:, : k * 2].reshape(h * w, k, 2)
    sigmas_raw = params[:, k * 2:].reshape(h * w, k)
    means = mids + _MMULT * means
    s = hw - 1.0
    means = jnp.remainder(means, s)
    sigmas = (jax.nn.softplus(sigmas_raw + _SIGMA_BOOST) + _MIN_SIGMA)[..., None] * hw
    sigmas = sigmas * _SIGMA_SCALE

    fl = jnp.floor(jax.lax.stop_gradient(means)).astype(jnp.int32)
    offs = jnp.array([[0, 0], [0, 1], [1, 0], [1, 1]], dtype=jnp.int32)
    neigh = fl[:, :, None, :] + offs[None, None]
    neigh = jnp.broadcast_to(neigh[None], (b, h * w, k, 4, 2))
    glob, roff = _random_index_constants(b, h, w, k)
    glob = glob.reshape(b, h * w, k, _GADD, 2)
    roff = roff.reshape(b, h * w, k, _RADD, 2)
    rel = fl[None, :, :, None, :] + roff
    bounds = jnp.array([h, w], dtype=jnp.int32)
    indices = jnp.concatenate([neigh, glob, rel], axis=3)
    indices = jnp.remainder(indices, bounds)

    indfl = indices.astype(jnp.float32)
    eq = jnp.all(indices[:, :, :, :, None, :] == indices[:, :, :, None, :, :], axis=-1)
    tril = jnp.tril(jnp.ones((_VS, _VS), dtype=bool), -1)
    dups = jnp.any(eq & tril[None, None, None], axis=-1)

    diff = (indfl - means[None, :, :, None, :]) / sigmas[None, :, :, None, :]
    props = jnp.exp(-0.5 * jnp.sum(diff * diff, axis=-1))
    props = jnp.where(dups, 0.0, props)
    weights = props / jnp.sum(props, axis=3, keepdims=True)

    lin = indices[..., 0] * w + indices[..., 1]
    rowid = lin + (jnp.arange(b, dtype=jnp.int32) * (h * w))[:, None, None, None]
    idxs = rowid.reshape(_NW, _NBLK, 128)
    wts = weights.reshape(_NW, _NBLK, 128)

    xt = x.transpose(0, 2, 3, 1).reshape(b * h * w, c)

    feat = _sc_gather_combine(xt, idxs, wts)
    feat = feat.reshape(b * h * w, k * c)

    out = _unify_matmul(Wu, feat, bu)
    return out.reshape(b, _COUT, h, w)


# speedup vs baseline: 1.2151x; 1.2151x over previous
import functools

import jax
import jax.numpy as jnp
import numpy as np
from jax import lax
from jax.experimental import pallas as pl
from jax.experimental.pallas import tpu as pltpu
from jax.experimental.pallas import tpu_sc as plsc

_B, _CIN, _H, _W = 4, 128, 32, 32
_COUT = 512
_K = 9
_GADD, _RADD = 2, 2
_REGION = (8, 8)
_MIN_SIGMA = 0.05
_SIGMA_SCALE = 0.05
_SIGMA_BOOST = 2.0
_MMULT = 1.0
_HIDDEN = _CIN * 4
_VS = 4 + _GADD + _RADD

_NW = 32
_G = _B * _H * _W * _K
_GPW = _G // _NW
_BG = 16
_NBLK = _GPW // _BG
_NROW = _B * _H * _W

_INTERPRET = False


def _coords_hw(h, w):
    ci = jnp.arange(h, dtype=jnp.float32) / (h - 1)
    cj = jnp.arange(w, dtype=jnp.float32) / (w - 1)
    return jnp.stack([
        jnp.broadcast_to(ci[:, None], (h, w)),
        jnp.broadcast_to(cj[None, :], (h, w)),
    ], axis=0)


def _random_index_constants(b, h, w, k):
    kg = jax.random.key(42)
    kg1, kg2 = jax.random.split(kg)
    glob = jnp.stack([
        jax.random.randint(jax.random.fold_in(kg1, 0), (b, h, w, k, _GADD), 0, h),
        jax.random.randint(jax.random.fold_in(kg1, 1), (b, h, w, k, _GADD), 0, w),
    ], axis=-1).astype(jnp.int32)
    roff = jnp.stack([
        jax.random.randint(jax.random.fold_in(kg2, 0), (b, h, w, k, _RADD), 0, _REGION[0]) - _REGION[0] // 2,
        jax.random.randint(jax.random.fold_in(kg2, 1), (b, h, w, k, _RADD), 0, _REGION[1]) - _REGION[1] // 2,
    ], axis=-1).astype(jnp.int32)
    return glob, roff


_GLOB_NP, _ROFF_NP = (np.asarray(a) for a in
                      _random_index_constants(_B, _H, _W, _K))


def _bcast_lane(vec, lane):
    idx = jnp.full((16, 1), lane, jnp.int32)
    dn = lax.GatherDimensionNumbers(
        offset_dims=(), collapsed_slice_dims=(0,), start_index_map=(0,))
    return lax.gather(vec, idx, dn, (1,),
                      mode=lax.GatherScatterMode.PROMISE_IN_BOUNDS)


def _sc_gather_combine(xt, idxs, wts):
    @functools.partial(
        pl.kernel,
        out_type=jax.ShapeDtypeStruct((_G, _CIN), jnp.float32),
        mesh=plsc.VectorSubcoreMesh(core_axis_name="c", subcore_axis_name="s"),
        scratch_types=[
            pltpu.VMEM((_NBLK, 128), jnp.int32),
            pltpu.VMEM((_NBLK, 128), jnp.float32),
            pltpu.VMEM((2, 128, _CIN), jnp.float32),
            pltpu.VMEM((2, _BG, _CIN), jnp.float32),
            pltpu.SemaphoreType.DMA,
            pltpu.SemaphoreType.DMA,
            pltpu.SemaphoreType.DMA,
            pltpu.SemaphoreType.DMA,
        ],
    )
    def k(xt_hbm, idx_hbm, w_hbm, feat_hbm, idx_v, w_v, rows_v, out_v,
          gsem0, gsem1, osem0, osem1):
        wid = lax.axis_index("s") * 2 + lax.axis_index("c")
        pltpu.sync_copy(idx_hbm.at[wid], idx_v)
        pltpu.sync_copy(w_hbm.at[wid], w_v)

        def combine(buf, blk):
            def gbody(pr, carry):
                wvec = w_v[blk, pl.ds(pr * 16, 16)]
                for half in range(2):
                    row0 = (pr * 2 + half) * _VS
                    wb = [_bcast_lane(wvec, half * _VS + vs) for vs in range(_VS)]
                    for cp in range(_CIN // 16):
                        acc = None
                        for vs in range(_VS):
                            r = rows_v[buf, row0 + vs, pl.ds(cp * 16, 16)]
                            acc = wb[vs] * r if acc is None else acc + wb[vs] * r
                        out_v[buf, pr * 2 + half, pl.ds(cp * 16, 16)] = acc
                return carry
            lax.fori_loop(0, _BG // 2, gbody, 0)

        def gather(buf, blk, sem):
            return pltpu.async_copy(xt_hbm.at[idx_v.at[blk]], rows_v.at[buf], sem)

        def wait_gather(buf, sem):
            pltpu.make_async_copy(xt_hbm.at[idx_v.at[0]], rows_v.at[buf], sem).wait()

        def put_out(buf, blk, sem):
            base = pl.multiple_of(wid * _GPW + blk * _BG, 8)
            return pltpu.async_copy(out_v.at[buf],
                                    feat_hbm.at[pl.ds(base, _BG)], sem)

        def wait_out(buf, sem):
            pltpu.make_async_copy(out_v.at[buf],
                                  feat_hbm.at[pl.ds(0, _BG)], sem).wait()

        gather(0, 0, gsem0)

        def body(i, carry):
            blk0 = i * 2
            blk1 = i * 2 + 1
            gather(1, blk1, gsem1)
            wait_gather(0, gsem0)

            @pl.when(i >= 1)
            def _():
                wait_out(0, osem0)
            combine(0, blk0)
            put_out(0, blk0, osem0)

            @pl.when(i + 1 < _NBLK // 2)
            def _():
                gather(0, blk0 + 2, gsem0)
            wait_gather(1, gsem1)

            @pl.when(i >= 1)
            def _():
                wait_out(1, osem1)
            combine(1, blk1)
            put_out(1, blk1, osem1)
            return carry

        lax.fori_loop(0, _NBLK // 2, body, 0)
        wait_out(0, osem0)
        wait_out(1, osem1)

    return k(xt, idxs, wts)


def _matmul_kernel(wu_ref, feat_ref, bu_ref, out_ref):
    acc = lax.dot_general(wu_ref[...], feat_ref[...],
                          (((1,), (1,)), ((), ())),
                          preferred_element_type=jnp.float32)
    out_ref[...] = (acc + bu_ref[...])[None]


def _unify_matmul(wu_b, feat, bu):
    cout, kc = wu_b.shape
    p = _H * _W
    grid = (_B,)
    return pl.pallas_call(
        _matmul_kernel,
        grid=grid,
        in_specs=[
            pl.BlockSpec((cout, kc), lambda i: (0, 0)),
            pl.BlockSpec((p, kc), lambda i: (i, 0)),
            pl.BlockSpec((cout, 1), lambda i: (0, 0)),
        ],
        out_specs=pl.BlockSpec((1, cout, p), lambda i: (i, 0, 0)),
        out_shape=jax.ShapeDtypeStruct((_B, cout, p), jnp.float32),
        interpret=_INTERPRET,
    )(wu_b, feat, bu.reshape(cout, 1))


def kernel(x, W1, b1, W2, b2, Wu, bu):
    b, c, h, w = x.shape
    k = _K
    coords = _coords_hw(h, w)
    hw = jnp.array([h, w], dtype=jnp.float32)

    mids = coords * (hw - 1.0)[:, None, None]
    mids = mids.transpose(1, 2, 0).reshape(h * w, 1, 2)

    inp = coords.transpose(1, 2, 0).reshape(h * w, 2)
    hdn = jax.nn.relu(inp @ W1.T + b1)
    params = hdn @ W2.T + b2
    means = params[:, : k * 2].reshape(h * w, k, 2)
    sigmas_raw = params[:, k * 2:].reshape(h * w, k)
    means = mids + _MMULT * means
    s = hw - 1.0
    means = jnp.remainder(means, s)
    sigmas = (jax.nn.softplus(sigmas_raw + _SIGMA_BOOST) + _MIN_SIGMA)[..., None] * hw
    sigmas = sigmas * _SIGMA_SCALE

    fl = jnp.floor(jax.lax.stop_gradient(means)).astype(jnp.int32)
    offs = jnp.array([[0, 0], [0, 1], [1, 0], [1, 1]], dtype=jnp.int32)
    neigh = fl[:, :, None, :] + offs[None, None]
    neigh = jnp.broadcast_to(neigh[None], (b, h * w, k, 4, 2))
    glob = jnp.asarray(_GLOB_NP.reshape(b, h * w, k, _GADD, 2))
    roff = jnp.asarray(_ROFF_NP.reshape(b, h * w, k, _RADD, 2))
    rel = fl[None, :, :, None, :] + roff
    bounds = jnp.array([h, w], dtype=jnp.int32)
    indices = jnp.concatenate([neigh, glob, rel], axis=3)
    indices = jnp.remainder(indices, bounds)

    indfl = indices.astype(jnp.float32)
    eq = jnp.all(indices[:, :, :, :, None, :] == indices[:, :, :, None, :, :], axis=-1)
    tril = jnp.tril(jnp.ones((_VS, _VS), dtype=bool), -1)
    dups = jnp.any(eq & tril[None, None, None], axis=-1)

    diff = (indfl - means[None, :, :, None, :]) / sigmas[None, :, :, None, :]
    props = jnp.exp(-0.5 * jnp.sum(diff * diff, axis=-1))
    props = jnp.where(dups, 0.0, props)
    weights = props / jnp.sum(props, axis=3, keepdims=True)

    lin = indices[..., 0] * w + indices[..., 1]
    rowid = lin + (jnp.arange(b, dtype=jnp.int32) * (h * w))[:, None, None, None]
    idxs = rowid.reshape(_NW, _NBLK, 128)
    wts = weights.reshape(_NW, _NBLK, 128)

    xt = x.transpose(0, 2, 3, 1).reshape(b * h * w, c)

    feat = _sc_gather_combine(xt, idxs, wts)
    feat = feat.reshape(b * h * w, k * c)

    out = _unify_matmul(Wu, feat, bu)
    return out.reshape(b, _COUT, h, w)
